# baseline (speedup 1.0000x reference)
import functools

import jax
import jax.numpy as jnp
from jax import lax
from jax.experimental import pallas as pl
from jax.experimental.pallas import tpu as pltpu
from jax.experimental.pallas import tpu_sc as plsc

NC = 2
NS = 16
L = 16

C = 16
D = 128
VPR = D // L


def _matmul_body(x_ref, w_ref, o_ref):
    o_ref[...] = jnp.dot(x_ref[...], w_ref[...],
                         preferred_element_type=jnp.float32)


def _tc_matmul(x, w, block_rows=2000):
    n, d_in = x.shape
    d_out = w.shape[1]
    grid = (n // block_rows,)
    return pl.pallas_call(
        _matmul_body,
        grid=grid,
        in_specs=[
            pl.BlockSpec((block_rows, d_in), lambda i: (i, 0)),
            pl.BlockSpec((d_in, d_out), lambda i: (0, 0)),
        ],
        out_specs=pl.BlockSpec((block_rows, d_out), lambda i: (i, 0)),
        out_shape=jax.ShapeDtypeStruct((n, d_out), jnp.float32),
    )(x, w)


def _combine_body(p_ref, o_ref):
    o_ref[...] = jnp.maximum(p_ref[0] + p_ref[1], 0.0)


def _tc_combine(partials, block_rows=2000):
    _, n, d = partials.shape
    grid = (n // block_rows,)
    return pl.pallas_call(
        _combine_body,
        grid=grid,
        in_specs=[pl.BlockSpec((2, block_rows, d), lambda i: (0, i, 0))],
        out_specs=pl.BlockSpec((block_rows, d), lambda i: (i, 0)),
        out_shape=jax.ShapeDtypeStruct((n, d), jnp.float32),
    )(partials)


NBUF = 5
PREF = 3


def _make_sc_agg(n, e):
    epw = e // (NC * NS)
    nch = epw // C
    rows_per_sub = 624
    zrows = 104
    nz = rows_per_sub // zrows
    tail_start = NS * rows_per_sub
    tail_rows = n - tail_start
    groups = C // L

    mesh = plsc.VectorSubcoreMesh(
        core_axis_name="c", subcore_axis_name="s",
        num_cores=NC, num_subcores=NS)

    @functools.partial(
        pl.kernel,
        out_type=jax.ShapeDtypeStruct((NC, n, D), jnp.float32),
        mesh=mesh,
        scratch_types=[
            [pltpu.VMEM((C,), jnp.int32) for _ in range(NBUF)],
            [pltpu.VMEM((C,), jnp.int32) for _ in range(NBUF)],
            [pltpu.VMEM((C,), jnp.float32) for _ in range(NBUF)],
            [pltpu.VMEM((C, D), jnp.float32) for _ in range(NBUF)],
            pltpu.VMEM((zrows, D), jnp.float32),
            pltpu.VMEM_SHARED((n, D), jnp.float32),
            [pltpu.SemaphoreType.DMA for _ in range(NBUF)],
            [pltpu.SemaphoreType.DMA for _ in range(NBUF)],
            pltpu.SemaphoreType.DMA,
        ],
    )
    def agg(pre_hbm, src_hbm, dst_hbm, adj_hbm, out_hbm,
            srcb, dstb, adjb, rows, zbuf, accum, gsem, lsem, ssem):
        c = lax.axis_index("c")
        s = lax.axis_index("s")
        wid = s * NC + c
        base = wid * epw

        def load_idx(g, k):
            off = pl.multiple_of(base + g * C, 8)
            pltpu.async_copy(src_hbm.at[pl.ds(off, C)], srcb[k], lsem[k])
            pltpu.async_copy(dst_hbm.at[pl.ds(off, C)], dstb[k], lsem[k])
            pltpu.async_copy(adj_hbm.at[pl.ds(off, C)], adjb[k], lsem[k])

        def wait_idx(k):
            pltpu.make_async_copy(src_hbm.at[pl.ds(0, C)], srcb[k],
                                  lsem[k]).wait()
            pltpu.make_async_copy(dst_hbm.at[pl.ds(0, C)], dstb[k],
                                  lsem[k]).wait()
            pltpu.make_async_copy(adj_hbm.at[pl.ds(0, C)], adjb[k],
                                  lsem[k]).wait()

        def start_gather(k):
            pltpu.async_copy(pre_hbm.at[srcb[k]], rows[k], gsem[k])

        def wait_gather(k):
            pltpu.make_async_copy(pre_hbm.at[srcb[k]], rows[k],
                                  gsem[k]).wait()

        def drain_scatter(k):
            pltpu.make_async_copy(rows[k], accum.at[dstb[k]], ssem).wait()

        for k in range(PREF + 1):
            load_idx(k, k)
        for k in range(PREF):
            wait_idx(k)
            start_gather(k)

        def zrow(i, _):
            for j in range(VPR):
                zbuf[i, pl.ds(j * L, L)] = jnp.zeros((L,), jnp.float32)
            return _
        lax.fori_loop(0, zrows, zrow, None)
        for k in range(nz):
            r0 = s * rows_per_sub + k * zrows
            pltpu.sync_copy(zbuf, accum.at[pl.ds(r0, zrows)])

        @pl.when(s == 0)
        def _():
            pltpu.sync_copy(zbuf.at[pl.ds(0, tail_rows)],
                            accum.at[pl.ds(tail_start, tail_rows)])
        plsc.subcore_barrier()

        def outer(o, _):
            for k in range(NBUF):
                g = o * NBUF + k
                wait_gather(k)

                @pl.when(g >= 2)
                def _():
                    drain_scatter(k)

                kg = (k + PREF) % NBUF

                @pl.when(g < nch - PREF)
                def _():
                    wait_idx(kg)
                    start_gather(kg)

                @pl.when(g < nch - PREF - 1)
                def _():
                    load_idx(g + PREF + 1, (k + PREF + 1) % NBUF)

                av = adjb[k][...]
                for u in range(C):
                    a = av[u]
                    for j in range(VPR):
                        sl = pl.ds(j * L, L)
                        rows[k][u, sl] = rows[k][u, sl] * a

                idxv = dstb[k][...]
                pltpu.async_copy(rows[k], accum.at[idxv], ssem, add=True)
            return _
        lax.fori_loop(0, nch // NBUF, outer, None)
        drain_scatter(0)
        drain_scatter(1)
        plsc.subcore_barrier()

        for k in range(nz):
            r0 = s * rows_per_sub + k * zrows
            pltpu.sync_copy(accum.at[pl.ds(r0, zrows)],
                            out_hbm.at[c, pl.ds(r0, zrows)])

        @pl.when(s == 0)
        def _():
            pltpu.sync_copy(accum.at[pl.ds(tail_start, tail_rows)],
                            out_hbm.at[c, pl.ds(tail_start, tail_rows)])

    return agg


def kernel(x, edge_index, adj_values, weights):
    n = x.shape[0]
    e = edge_index.shape[1]
    pre_sup = _tc_matmul(x, weights)
    src = edge_index[0]
    dst = edge_index[1]
    partials = _make_sc_agg(n, e)(pre_sup, src, dst, adj_values)
    return _tc_combine(partials)


# speedup vs baseline: 2.0794x; 2.0794x over previous
import functools

import jax
import jax.numpy as jnp
from jax import lax
from jax.experimental import pallas as pl
from jax.experimental.pallas import tpu as pltpu
from jax.experimental.pallas import tpu_sc as plsc

NC = 2
NS = 16
L = 16

C = 64
D = 128
VPR = D // L


def _matmul_body(x_ref, w_ref, o_ref):
    o_ref[...] = jnp.dot(x_ref[...], w_ref[...],
                         preferred_element_type=jnp.float32)


def _tc_matmul(x, w, block_rows=2000):
    n, d_in = x.shape
    d_out = w.shape[1]
    grid = (n // block_rows,)
    return pl.pallas_call(
        _matmul_body,
        grid=grid,
        in_specs=[
            pl.BlockSpec((block_rows, d_in), lambda i: (i, 0)),
            pl.BlockSpec((d_in, d_out), lambda i: (0, 0)),
        ],
        out_specs=pl.BlockSpec((block_rows, d_out), lambda i: (i, 0)),
        out_shape=jax.ShapeDtypeStruct((n, d_out), jnp.float32),
    )(x, w)


def _combine_body(p_ref, o_ref):
    o_ref[...] = jnp.maximum(p_ref[0] + p_ref[1], 0.0)


def _tc_combine(partials, block_rows=2000):
    _, n, d = partials.shape
    grid = (n // block_rows,)
    return pl.pallas_call(
        _combine_body,
        grid=grid,
        in_specs=[pl.BlockSpec((2, block_rows, d), lambda i: (0, i, 0))],
        out_specs=pl.BlockSpec((block_rows, d), lambda i: (i, 0)),
        out_shape=jax.ShapeDtypeStruct((n, d), jnp.float32),
    )(partials)


NBUF = 4
PREF = 2


def _make_sc_agg(n, e):
    epw = e // (NC * NS)
    nch = epw // C
    tail = epw - nch * C
    rows_per_sub = 624
    zrows = 104
    nz = rows_per_sub // zrows
    tail_start = NS * rows_per_sub
    tail_rows = n - tail_start
    groups = C // L

    mesh = plsc.VectorSubcoreMesh(
        core_axis_name="c", subcore_axis_name="s",
        num_cores=NC, num_subcores=NS)

    @functools.partial(
        pl.kernel,
        out_type=jax.ShapeDtypeStruct((NC, n, D), jnp.float32),
        mesh=mesh,
        scratch_types=[
            [pltpu.VMEM((C,), jnp.int32) for _ in range(NBUF)],
            [pltpu.VMEM((C,), jnp.int32) for _ in range(NBUF)],
            [pltpu.VMEM((C,), jnp.float32) for _ in range(NBUF)],
            [pltpu.VMEM((C, D), jnp.float32) for _ in range(NBUF)],
            pltpu.VMEM((zrows, D), jnp.float32),
            pltpu.VMEM_SHARED((n, D), jnp.float32),
            [pltpu.SemaphoreType.DMA for _ in range(NBUF)],
            [pltpu.SemaphoreType.DMA for _ in range(NBUF)],
            pltpu.SemaphoreType.DMA,
        ],
    )
    def agg(pre_hbm, src_hbm, dst_hbm, adj_hbm, out_hbm,
            srcb, dstb, adjb, rows, zbuf, accum, gsem, lsem, ssem):
        c = lax.axis_index("c")
        s = lax.axis_index("s")
        wid = s * NC + c
        base = wid * epw

        def load_idx(g, k):
            off = pl.multiple_of(base + g * C, 8)
            pltpu.async_copy(src_hbm.at[pl.ds(off, C)], srcb[k], lsem[k])
            pltpu.async_copy(dst_hbm.at[pl.ds(off, C)], dstb[k], lsem[k])
            pltpu.async_copy(adj_hbm.at[pl.ds(off, C)], adjb[k], lsem[k])

        def wait_idx(k):
            pltpu.make_async_copy(src_hbm.at[pl.ds(0, C)], srcb[k],
                                  lsem[k]).wait()
            pltpu.make_async_copy(dst_hbm.at[pl.ds(0, C)], dstb[k],
                                  lsem[k]).wait()
            pltpu.make_async_copy(adj_hbm.at[pl.ds(0, C)], adjb[k],
                                  lsem[k]).wait()

        def start_gather(k):
            pltpu.async_copy(pre_hbm.at[srcb[k]], rows[k], gsem[k])

        def wait_gather(k):
            pltpu.make_async_copy(pre_hbm.at[srcb[k]], rows[k],
                                  gsem[k]).wait()

        def drain_scatter(k):
            pltpu.make_async_copy(rows[k], accum.at[dstb[k]], ssem).wait()

        for k in range(PREF + 1):
            load_idx(k, k)
        for k in range(PREF):
            wait_idx(k)
            start_gather(k)

        def scale_rows(rows_k, adj_k, ngroups):
            def group(gr, _):
                av = adj_k[pl.ds(gr * L, L)]
                for u in range(L):
                    a = av[u]
                    for j in range(VPR):
                        sl = pl.ds(j * L, L)
                        rows_k[gr * L + u, sl] = rows_k[gr * L + u, sl] * a
                return _
            lax.fori_loop(0, ngroups, group, None)

        def zrow(i, _):
            for j in range(VPR):
                zbuf[i, pl.ds(j * L, L)] = jnp.zeros((L,), jnp.float32)
            return _
        lax.fori_loop(0, zrows, zrow, None)
        for k in range(nz):
            r0 = s * rows_per_sub + k * zrows
            pltpu.sync_copy(zbuf, accum.at[pl.ds(r0, zrows)])

        @pl.when(s == 0)
        def _():
            pltpu.sync_copy(zbuf.at[pl.ds(0, tail_rows)],
                            accum.at[pl.ds(tail_start, tail_rows)])
        plsc.subcore_barrier()

        def outer(o, _):
            for k in range(NBUF):
                g = o * NBUF + k
                wait_gather(k)

                @pl.when(g >= 2)
                def _():
                    drain_scatter(k)

                kg = (k + PREF) % NBUF

                @pl.when(g < nch - PREF)
                def _():
                    wait_idx(kg)
                    start_gather(kg)

                @pl.when(g < nch - PREF - 1)
                def _():
                    load_idx(g + PREF + 1, (k + PREF + 1) % NBUF)

                scale_rows(rows[k], adjb[k], C // L)

                def scat(gr, _):
                    idxv = dstb[k][pl.ds(gr * L, L)]
                    pltpu.async_copy(rows[k].at[pl.ds(gr * L, L)],
                                     accum.at[idxv], ssem, add=True)
                    return _
                lax.fori_loop(0, C // L, scat, None)
            return _
        lax.fori_loop(0, nch // NBUF, outer, None)
        drain_scatter(0)
        drain_scatter(1)

        if tail:
            off = pl.multiple_of(base + nch * C, 8)
            pltpu.sync_copy(src_hbm.at[pl.ds(off, tail)],
                            srcb[0].at[pl.ds(0, tail)])
            pltpu.sync_copy(dst_hbm.at[pl.ds(off, tail)],
                            dstb[0].at[pl.ds(0, tail)])
            pltpu.sync_copy(adj_hbm.at[pl.ds(off, tail)],
                            adjb[0].at[pl.ds(0, tail)])
            pltpu.async_copy(pre_hbm.at[srcb[0].at[pl.ds(0, tail)]],
                             rows[0].at[pl.ds(0, tail)], gsem[0])
            wait_gather_tail = pltpu.make_async_copy(
                pre_hbm.at[srcb[0].at[pl.ds(0, tail)]],
                rows[0].at[pl.ds(0, tail)], gsem[0])
            wait_gather_tail.wait()
            scale_rows(rows[0], adjb[0], tail // L)

            def tscat(gr, _):
                idxv = dstb[0][pl.ds(gr * L, L)]
                pltpu.sync_copy(rows[0].at[pl.ds(gr * L, L)],
                                accum.at[idxv], add=True)
                return _
            lax.fori_loop(0, tail // L, tscat, None)
        plsc.subcore_barrier()

        for k in range(nz):
            r0 = s * rows_per_sub + k * zrows
            pltpu.sync_copy(accum.at[pl.ds(r0, zrows)],
                            out_hbm.at[c, pl.ds(r0, zrows)])

        @pl.when(s == 0)
        def _():
            pltpu.sync_copy(accum.at[pl.ds(tail_start, tail_rows)],
                            out_hbm.at[c, pl.ds(tail_start, tail_rows)])

    return agg


def kernel(x, edge_index, adj_values, weights):
    n = x.shape[0]
    e = edge_index.shape[1]
    pre_sup = _tc_matmul(x, weights)
    src = edge_index[0]
    dst = edge_index[1]
    partials = _make_sc_agg(n, e)(pre_sup, src, dst, adj_values)
    return _tc_combine(partials)
